# baseline (speedup 1.0000x reference)
import functools

import numpy as np

import jax
import jax.numpy as jnp
from jax import lax
from jax.experimental import pallas as pl
from jax.experimental.pallas import tpu as pltpu
from jax.experimental.pallas import tpu_sc as plsc

N = 10000
F = 128
E = 320000
T = 2

NC = 2
NS = 16
NW = NC * NS
EPW = E // NW
CH = 128
NCHP = -(-EPW // CH)
SEG = 2
CPS = 40
EPWP = SEG * CPS * CH
EXP_TILES = 10
RPT = N // EXP_TILES

_HI_MASK = -65536


def _sc_spmm(xs0, xs1, dst4, src4, w4):
    mesh = plsc.VectorSubcoreMesh(core_axis_name="c", subcore_axis_name="s")

    @functools.partial(
        pl.kernel,
        out_type=jax.ShapeDtypeStruct((T * NC * N, F), jnp.float32),
        mesh=mesh,
        scratch_types=[
            pltpu.VMEM_SHARED((N, F), jnp.float32),
            pltpu.VMEM((CPS, CH), jnp.int32),
            pltpu.VMEM((CPS, CH), jnp.int32),
            pltpu.VMEM((CPS, CH), jnp.float32),
            pltpu.VMEM((CH, F // 2), jnp.int32),
            pltpu.VMEM((CH, F // 2), jnp.int32),
            pltpu.VMEM((CH, F), jnp.float32),
            pltpu.SemaphoreType.DMA,
            pltpu.SemaphoreType.DMA,
        ],
        compiler_params=pltpu.CompilerParams(needs_layout_passes=False,
                                             use_tc_tiling_on_sc=False),
    )
    def kern(x0_hbm, x1_hbm, dst_hbm, src_hbm, w_hbm, out_hbm,
             acc, dstb, srcb, wb, gb0, gb1, sbuf, gsem0, gsem1):
        c = lax.axis_index("c")
        s = lax.axis_index("s")
        wid = s * NC + c
        gbufs = (gb0, gb1)
        gsems = (gsem0, gsem1)

        zero16 = jnp.zeros((16,), jnp.float32)

        def scale_chunk(gb, i):
            def egroup(g, gcarry):
                wv = wb[i, pl.ds(g * 16, 16)]
                for j in range(16):
                    e = g * 16 + j
                    bc = jnp.full((16,), wv[j], jnp.float32)
                    for k in range(F // 32):
                        u = gb[e, pl.ds(k * 16, 16)]
                        lo = plsc.bitcast(u << 16, jnp.float32)
                        hi = plsc.bitcast(u & _HI_MASK, jnp.float32)
                        sbuf[e, pl.ds(k * 32, 16)] = lo * bc
                        sbuf[e, pl.ds(k * 32 + 16, 16)] = hi * bc
                return gcarry

            lax.fori_loop(0, CH // 16, egroup, 0)

        for t in range(T):
            x_hbm = x0_hbm if t == 0 else x1_hbm

            def zrow(r, carry):
                for k in range(F // 16):
                    sbuf[r, pl.ds(k * 16, 16)] = zero16
                return carry

            lax.fori_loop(0, CH, zrow, 0)

            @pl.when(s < EXP_TILES)
            def _clear():
                for j in range(7):
                    pltpu.sync_copy(
                        sbuf, acc.at[pl.ds(s * RPT + j * CH, CH)])
                pltpu.sync_copy(sbuf.at[pl.ds(0, RPT - 7 * CH)],
                                acc.at[pl.ds(s * RPT + 7 * CH,
                                             RPT - 7 * CH)])

            plsc.subcore_barrier()

            for seg in range(SEG):
                pltpu.sync_copy(dst_hbm.at[wid, seg], dstb)
                pltpu.sync_copy(src_hbm.at[wid, seg], srcb)
                pltpu.sync_copy(w_hbm.at[wid, seg], wb)

                pltpu.async_copy(x_hbm.at[srcb.at[0]], gb0, gsem0)
                pltpu.async_copy(x_hbm.at[srcb.at[1]], gb1, gsem1)

                def do_chunk(i, b, prefetch):
                    gb, gsem = gbufs[b], gsems[b]
                    pltpu.make_async_copy(x_hbm.at[srcb.at[i]], gb,
                                          gsem).wait()
                    scale_chunk(gb, i)
                    if prefetch:
                        pltpu.async_copy(x_hbm.at[srcb.at[i + 2]], gb,
                                         gsem)
                    pltpu.sync_copy(sbuf, acc.at[dstb.at[i]], add=True)

                def pair(ii, carry):
                    do_chunk(2 * ii, 0, True)
                    do_chunk(2 * ii + 1, 1, True)
                    return carry

                lax.fori_loop(0, CPS // 2 - 1, pair, 0)
                do_chunk(CPS - 2, 0, False)
                do_chunk(CPS - 1, 1, False)

            plsc.subcore_barrier()

            @pl.when(s < EXP_TILES)
            def _export():
                base = (t * NC + c) * N + s * RPT
                pltpu.sync_copy(acc.at[pl.ds(s * RPT, RPT)],
                                out_hbm.at[pl.ds(base, RPT)])

            plsc.subcore_barrier()

    return kern(xs0, xs1, dst4, src4, w4)


def _cast_body(x, out):
    out[0, 0] = x[0, 0].astype(jnp.bfloat16)


def _cast_to_bf16(inputs):
    bn = 2000
    return pl.pallas_call(
        _cast_body,
        grid=(T, N // bn),
        in_specs=[pl.BlockSpec((1, 1, bn, F), lambda t, i: (0, t, i, 0))],
        out_specs=pl.BlockSpec((1, 1, bn, F), lambda t, i: (t, 0, i, 0)),
        out_shape=jax.ShapeDtypeStruct((T, 1, N, F), jnp.bfloat16),
    )(inputs)


def _tc_body(pa, pb, w, out):
    a = pa[0, 0] + pb[0, 0]
    y = lax.dot(a, w[...], precision=lax.Precision.HIGHEST,
                preferred_element_type=jnp.float32)
    out[0, 0] = jnp.maximum(y, 0.0)


def _tc_transform(p, w0):
    bn = 2000
    grid = (T, N // bn)
    return pl.pallas_call(
        _tc_body,
        grid=grid,
        in_specs=[
            pl.BlockSpec((1, 1, bn, F), lambda t, i: (t, 0, i, 0)),
            pl.BlockSpec((1, 1, bn, F), lambda t, i: (t, 1, i, 0)),
            pl.BlockSpec((F, F), lambda t, i: (0, 0)),
        ],
        out_specs=pl.BlockSpec((1, 1, bn, F), lambda t, i: (0, t, i, 0)),
        out_shape=jax.ShapeDtypeStruct((1, T, N, F), jnp.float32),
    )(p, p, w0)


def _pad_edges(a, fill):
    a2 = a.reshape(NW, EPW)
    return jnp.pad(a2, ((0, 0), (0, EPWP - EPW)),
                   constant_values=fill).reshape(NW, SEG, CPS, CH)


_PERM = sum((list(range(k * 32, k * 32 + 32, 2))
             + list(range(k * 32 + 1, k * 32 + 32, 2))
             for k in range(F // 32)), [])


def kernel(inputs, edge_index, edge_weight, W0):
    xb = _cast_to_bf16(inputs).reshape(T, N, F // 2, 2)
    xs = lax.bitcast_convert_type(xb, jnp.int32)
    dst4 = _pad_edges(edge_index[0], 0)
    src4 = _pad_edges(edge_index[1], 0)
    w4 = _pad_edges(edge_weight, 0.0)

    partials = _sc_spmm(xs[0], xs[1], dst4, src4, w4)
    p = partials.reshape(T, NC, N, F)
    out = _tc_transform(p, W0[np.array(_PERM, dtype=np.int32)])
    return (out, W0)


# speedup vs baseline: 1.1109x; 1.1109x over previous
import functools

import jax
import jax.numpy as jnp
from jax import lax
from jax.experimental import pallas as pl
from jax.experimental.pallas import tpu as pltpu
from jax.experimental.pallas import tpu_sc as plsc

N = 10000
F = 128
E = 320000
T = 2

NC = 2
NS = 16
NW = NC * NS
EPW = E // NW
CH = 128
SEG = 2
CPS = 40
EPWP = SEG * CPS * CH
EXP_TILES = 10
RPT = N // EXP_TILES


def _sc_spmm(x0, x1, dst4, src4, w4):
    mesh = plsc.VectorSubcoreMesh(core_axis_name="c", subcore_axis_name="s")

    @functools.partial(
        pl.kernel,
        out_type=jax.ShapeDtypeStruct((T * NC * N, F), jnp.float32),
        mesh=mesh,
        scratch_types=[
            pltpu.VMEM_SHARED((N, F), jnp.float32),
            pltpu.VMEM((CPS, CH), jnp.int32),
            pltpu.VMEM((CPS, CH), jnp.int32),
            pltpu.VMEM((CPS, CH), jnp.float32),
            pltpu.VMEM((CH, F), jnp.float32),
            pltpu.VMEM((CH, F), jnp.float32),
            pltpu.SemaphoreType.DMA,
            pltpu.SemaphoreType.DMA,
        ],
    )
    def kern(x0_hbm, x1_hbm, dst_hbm, src_hbm, w_hbm, out_hbm,
             acc, dstb, srcb, wb, gb0, gb1, gsem0, gsem1):
        c = lax.axis_index("c")
        s = lax.axis_index("s")
        wid = s * NC + c
        gbufs = (gb0, gb1)
        gsems = (gsem0, gsem1)

        zero16 = jnp.zeros((16,), jnp.float32)

        def scale_chunk(gb, i):
            def egroup(g, gcarry):
                wv = wb[i, pl.ds(g * 16, 16)]
                for j in range(16):
                    e = g * 16 + j
                    bc = jnp.full((16,), wv[j], jnp.float32)
                    for k in range(F // 16):
                        gb[e, pl.ds(k * 16, 16)] = (
                            gb[e, pl.ds(k * 16, 16)] * bc)
                return gcarry

            lax.fori_loop(0, CH // 16, egroup, 0)

        for t in range(T):
            x_hbm = x0_hbm if t == 0 else x1_hbm

            def zrow(r, carry):
                for k in range(F // 16):
                    gb0[r, pl.ds(k * 16, 16)] = zero16
                return carry

            lax.fori_loop(0, CH, zrow, 0)

            @pl.when(s < EXP_TILES)
            def _clear():
                for j in range(7):
                    pltpu.sync_copy(
                        gb0, acc.at[pl.ds(s * RPT + j * CH, CH)])
                pltpu.sync_copy(gb0.at[pl.ds(0, RPT - 7 * CH)],
                                acc.at[pl.ds(s * RPT + 7 * CH,
                                             RPT - 7 * CH)])

            plsc.subcore_barrier()

            for seg in range(SEG):
                pltpu.sync_copy(dst_hbm.at[wid, seg], dstb)
                pltpu.sync_copy(src_hbm.at[wid, seg], srcb)
                pltpu.sync_copy(w_hbm.at[wid, seg], wb)

                pltpu.async_copy(x_hbm.at[srcb.at[0]], gb0, gsem0)
                pltpu.async_copy(x_hbm.at[srcb.at[1]], gb1, gsem1)

                def do_chunk(i, b, prefetch):
                    gb, gsem = gbufs[b], gsems[b]
                    pltpu.make_async_copy(x_hbm.at[srcb.at[i]], gb,
                                          gsem).wait()
                    scale_chunk(gb, i)
                    pltpu.sync_copy(gb, acc.at[dstb.at[i]], add=True)
                    if prefetch:
                        pltpu.async_copy(x_hbm.at[srcb.at[i + 2]], gb,
                                         gsem)

                def pair(ii, carry):
                    do_chunk(2 * ii, 0, True)
                    do_chunk(2 * ii + 1, 1, True)
                    return carry

                lax.fori_loop(0, CPS // 2 - 1, pair, 0)
                do_chunk(CPS - 2, 0, False)
                do_chunk(CPS - 1, 1, False)

            plsc.subcore_barrier()

            @pl.when(s < EXP_TILES)
            def _export():
                base = (t * NC + c) * N + s * RPT
                pltpu.sync_copy(acc.at[pl.ds(s * RPT, RPT)],
                                out_hbm.at[pl.ds(base, RPT)])

            plsc.subcore_barrier()

    return kern(x0, x1, dst4, src4, w4)


def _tc_body(pa, pb, w, out):
    a = pa[0, 0] + pb[0, 0]
    y = lax.dot(a, w[...], precision=lax.Precision.HIGHEST,
                preferred_element_type=jnp.float32)
    out[0, 0] = jnp.maximum(y, 0.0)


def _tc_transform(p, w0):
    bn = 2000
    grid = (T, N // bn)
    return pl.pallas_call(
        _tc_body,
        grid=grid,
        in_specs=[
            pl.BlockSpec((1, 1, bn, F), lambda t, i: (t, 0, i, 0)),
            pl.BlockSpec((1, 1, bn, F), lambda t, i: (t, 1, i, 0)),
            pl.BlockSpec((F, F), lambda t, i: (0, 0)),
        ],
        out_specs=pl.BlockSpec((1, 1, bn, F), lambda t, i: (0, t, i, 0)),
        out_shape=jax.ShapeDtypeStruct((1, T, N, F), jnp.float32),
    )(p, p, w0)


def _pad_edges(a, fill):
    a2 = a.reshape(NW, EPW)
    return jnp.pad(a2, ((0, 0), (0, EPWP - EPW)),
                   constant_values=fill).reshape(NW, SEG, CPS, CH)


def kernel(inputs, edge_index, edge_weight, W0):
    x0 = inputs[0, 0]
    x1 = inputs[0, 1]
    dst4 = _pad_edges(edge_index[0], 0)
    src4 = _pad_edges(edge_index[1], 0)
    w4 = _pad_edges(edge_weight, 0.0)

    partials = _sc_spmm(x0, x1, dst4, src4, w4)
    p = partials.reshape(T, NC, N, F)
    out = _tc_transform(p, W0)
    return (out, W0)
